# baseline (speedup 1.0000x reference)
import jax
import jax.numpy as jnp
from jax.experimental import pallas as pl
from jax.experimental.pallas import tpu as pltpu


def _make_kernel(n_hsteps, chunk, hm, n_msteps, nslots):
    def _kern(x_ref, w_ref, adjA_ref, adj_hbm, b_ref, out_ref,
              h_ref, ring, sem_a):
        i = pl.program_id(0)

        @pl.when(i < n_hsteps)
        def _():
            xb = x_ref[...].astype(jnp.bfloat16)
            wb = w_ref[...].astype(jnp.bfloat16)
            h_ref[pl.ds(i * chunk, chunk), :] = jnp.dot(
                xb, wb,
                preferred_element_type=jnp.float32).astype(jnp.bfloat16)

        @pl.when(i == n_hsteps - 1)
        def _():
            for s in range(nslots):
                pltpu.make_async_copy(
                    adj_hbm.at[pl.ds(s * 2 * hm + hm, hm), :],
                    ring.at[s],
                    sem_a.at[s],
                ).start()

        @pl.when(i >= n_hsteps)
        def _():
            j = i - n_hsteps

            def _step(s):
                def _br():
                    aA = adjA_ref[...].astype(jnp.bfloat16)
                    out_ref[:hm, :] = jnp.dot(
                        aA, h_ref[...],
                        preferred_element_type=jnp.float32) + b_ref[...]
                    pltpu.make_async_copy(
                        adj_hbm.at[pl.ds(j * 2 * hm + hm, hm), :],
                        ring.at[s],
                        sem_a.at[s],
                    ).wait()
                    aB = ring[s].astype(jnp.bfloat16)
                    out_ref[hm:, :] = jnp.dot(
                        aB, h_ref[...],
                        preferred_element_type=jnp.float32) + b_ref[...]

                    @pl.when(j + nslots < n_msteps)
                    def _():
                        pltpu.make_async_copy(
                            adj_hbm.at[pl.ds((j + nslots) * 2 * hm + hm, hm), :],
                            ring.at[s],
                            sem_a.at[s],
                        ).start()
                return _br

            jax.lax.switch(jax.lax.rem(j, nslots),
                           [_step(s) for s in range(nslots)])

    return _kern


def kernel(x, adj, W, b):
    n, f = x.shape
    h_dim = W.shape[1]

    n_hsteps = 5 if n % (5 * 8) == 0 else 1
    chunk = n // n_hsteps
    bm = 400 if n % 400 == 0 else n
    hm = bm // 2
    n_msteps = n // bm
    nslots = min(3, n_msteps)
    grid = (n_hsteps + n_msteps,)

    out = pl.pallas_call(
        _make_kernel(n_hsteps, chunk, hm, n_msteps, nslots),
        grid=grid,
        in_specs=[
            pl.BlockSpec((chunk, f), lambda i: (jnp.minimum(i, n_hsteps - 1), 0)),
            pl.BlockSpec((f, h_dim), lambda i: (0, 0)),
            pl.BlockSpec((hm, n), lambda i: (2 * jnp.maximum(i - n_hsteps, 0), 0)),
            pl.BlockSpec(memory_space=pltpu.MemorySpace.HBM),
            pl.BlockSpec((1, h_dim), lambda i: (0, 0)),
        ],
        out_specs=pl.BlockSpec((bm, h_dim), lambda i: (jnp.maximum(i - n_hsteps, 0), 0)),
        out_shape=jax.ShapeDtypeStruct((n, h_dim), jnp.float32),
        scratch_shapes=[
            pltpu.VMEM((n, h_dim), jnp.bfloat16),
            pltpu.VMEM((nslots, hm, n), jnp.float32),
            pltpu.SemaphoreType.DMA((nslots,)),
        ],
        compiler_params=pltpu.CompilerParams(
            dimension_semantics=("arbitrary",),
            vmem_limit_bytes=66 * 1024 * 1024,
        ),
    )(x, W, adj, adj, b.reshape(1, h_dim))
    return out


# speedup vs baseline: 1.0316x; 1.0316x over previous
import jax
import jax.numpy as jnp
from jax.experimental import pallas as pl
from jax.experimental.pallas import tpu as pltpu


def _make_kernel(n_hsteps, chunk):
    def _fused_kernel(x_ref, w_ref, adj_ref, b_ref, out_ref, h_ref):
        i = pl.program_id(0)

        @pl.when(i < n_hsteps)
        def _():
            xb = x_ref[...].astype(jnp.bfloat16)
            wb = w_ref[...].astype(jnp.bfloat16)
            h_ref[pl.ds(i * chunk, chunk), :] = jnp.dot(
                xb, wb,
                preferred_element_type=jnp.float32).astype(jnp.bfloat16)

        @pl.when(i >= n_hsteps)
        def _():
            a = adj_ref[...].astype(jnp.bfloat16)
            out_ref[...] = jnp.dot(
                a, h_ref[...],
                preferred_element_type=jnp.float32) + b_ref[...]

    return _fused_kernel


def kernel(x, adj, W, b):
    n, f = x.shape
    h_dim = W.shape[1]

    n_hsteps = 5 if n % (5 * 8) == 0 else 1
    chunk = n // n_hsteps
    bm = 400 if n % 400 == 0 else n
    n_msteps = n // bm
    grid = (n_hsteps + n_msteps,)

    out = pl.pallas_call(
        _make_kernel(n_hsteps, chunk),
        grid=grid,
        in_specs=[
            pl.BlockSpec((chunk, f), lambda i: (jnp.minimum(i, n_hsteps - 1), 0)),
            pl.BlockSpec((f, h_dim), lambda i: (0, 0)),
            pl.BlockSpec((bm, n), lambda i: (jnp.maximum(i - n_hsteps, 0), 0)),
            pl.BlockSpec((1, h_dim), lambda i: (0, 0)),
        ],
        out_specs=pl.BlockSpec((bm, h_dim), lambda i: (jnp.maximum(i - n_hsteps, 0), 0)),
        out_shape=jax.ShapeDtypeStruct((n, h_dim), jnp.float32),
        scratch_shapes=[pltpu.VMEM((n, h_dim), jnp.bfloat16)],
        compiler_params=pltpu.CompilerParams(
            dimension_semantics=("arbitrary",),
        ),
    )(x, W, adj, b.reshape(1, h_dim))
    return out
